# baseline (speedup 1.0000x reference)
import jax
import jax.numpy as jnp
from jax import lax
from jax.experimental import pallas as pl
from jax.experimental.pallas import tpu as pltpu
from jax.experimental.pallas import tpu_sc as plsc

N_WORDS = 1000
N_PHONES = 64
IGNORE_IDX = -100
LANES = 16
NUM_CORES = 2
NUM_SUBCORES = 16
NUM_WORKERS = NUM_CORES * NUM_SUBCORES
BATCH = 4096
SEQ = 50
TOTAL_IDX = BATCH * SEQ
BBLK = BATCH // NUM_WORKERS
IDX_PER_WORKER = BBLK * SEQ
ROWS_PAD = 1008
LBUF = N_PHONES * BBLK
TILE = 8 * 128
P_TILES = N_PHONES // 8
L_STRIDE = 8 * 32 * 8 * 128
P_STRIDE = 32 * 8 * 128


def _tile_body(x_hbm, tbl_hbm, out_hbm,
               tbl_v, rnorm_v, idx_raw, idx_t, out_v0, out_v1,
               isem, osem0, osem1):
    cid = lax.axis_index("c")
    sid = lax.axis_index("s")
    wid = cid * NUM_SUBCORES + sid
    iota = lax.iota(jnp.int32, LANES)

    idx_cp = pltpu.async_copy(
        x_hbm.at[pl.ds(wid * IDX_PER_WORKER, IDX_PER_WORKER)], idx_raw, isem)
    pltpu.sync_copy(tbl_hbm, tbl_v.at[pl.ds(0, N_WORDS * N_PHONES)])

    zeros16 = jnp.zeros((LANES,), jnp.float32)
    for k in range((ROWS_PAD - N_WORDS) * N_PHONES // LANES):
        tbl_v[pl.ds(N_WORDS * N_PHONES + k * LANES, LANES)] = zeros16

    perm = [(iota + t) & (LANES - 1) for t in range(LANES)]

    def norm_group(k, carry):
        rows64 = (k * LANES + iota) * N_PHONES
        acc = [jnp.zeros((LANES,), jnp.float32) for _ in range(4)]
        for q in range(N_PHONES // LANES):
            rq = rows64 + q * LANES
            vs = [plsc.load_gather(tbl_v, [rq + perm[t]])
                  for t in range(LANES)]
            for t in range(LANES):
                acc[t % 4] = acc[t % 4] + vs[t]
        s = (acc[0] + acc[1]) + (acc[2] + acc[3])
        s = jnp.where(s > 0.0, s, 1.0)
        rnorm_v[pl.ds(k * LANES, LANES)] = 1.0 / s
        return carry

    lax.fori_loop(0, ROWS_PAD // LANES, norm_group, 0)
    idx_cp.wait()

    tr_base = [(g * LANES + iota) * SEQ for g in range(BBLK // LANES)]

    def tr_step(l, carry):
        for g in range(BBLK // LANES):
            raw = plsc.load_gather(idx_raw, [tr_base[g] + l])
            rows = jnp.where(raw == IGNORE_IDX, 0, raw)
            idx_t[pl.ds(l * BBLK + g * LANES, LANES)] = rows
        return carry

    lax.fori_loop(0, SEQ, tr_step, 0)

    def compute_l(l, out_buf):
        def group(g, carry):
            rows = idx_t[pl.ds(l * BBLK + g * LANES, LANES)]
            rn = plsc.load_gather(rnorm_v, [rows])
            rows64 = rows * N_PHONES
            g16 = g * LANES
            for q in range(N_PHONES // LANES):
                rq = rows64 + q * LANES
                sb = iota + (q * LANES * BBLK + g16)
                for h in range(2):
                    ps = [(iota + (t + g16)) & (LANES - 1)
                          for t in range(h * 8, h * 8 + 8)]
                    vs = [plsc.load_gather(tbl_v, [rq + p]) for p in ps]
                    for p, v in zip(ps, vs):
                        plsc.store_scatter(out_buf, [p * BBLK + sb], v * rn)
            return carry

        lax.fori_loop(0, BBLK // LANES, group, 0)

    def start_out(l, out_buf, sem):
        for P in range(P_TILES):
            pltpu.async_copy(
                out_buf.at[pl.ds(P * TILE, TILE)],
                out_hbm.at[pl.ds(l * L_STRIDE + P * P_STRIDE + wid * TILE, TILE)],
                sem)

    def drain(out_buf, sem):
        pltpu.make_async_copy(out_hbm.at[pl.ds(0, LBUF)], out_buf, sem).wait()

    compute_l(0, out_v0)
    start_out(0, out_v0, osem0)
    compute_l(1, out_v1)
    start_out(1, out_v1, osem1)

    def pair(p, carry):
        l0 = 2 * p
        drain(out_v0, osem0)
        compute_l(l0, out_v0)
        start_out(l0, out_v0, osem0)
        drain(out_v1, osem1)
        compute_l(l0 + 1, out_v1)
        start_out(l0 + 1, out_v1, osem1)
        return carry

    lax.fori_loop(1, SEQ // 2, pair, 0)

    drain(out_v0, osem0)
    drain(out_v1, osem1)


def kernel(x, pron_counts):
    xf = x.reshape(-1).astype(jnp.int32)
    tblf = pron_counts.reshape(-1)
    mesh = plsc.VectorSubcoreMesh(
        core_axis_name="c", subcore_axis_name="s",
        num_cores=NUM_CORES, num_subcores=NUM_SUBCORES)
    out = pl.kernel(
        _tile_body,
        out_type=jax.ShapeDtypeStruct((TOTAL_IDX * N_PHONES,), jnp.float32),
        mesh=mesh,
        compiler_params=pltpu.CompilerParams(needs_layout_passes=False),
        scratch_types=[
            pltpu.VMEM((ROWS_PAD * N_PHONES,), jnp.float32),
            pltpu.VMEM((ROWS_PAD,), jnp.float32),
            pltpu.VMEM((IDX_PER_WORKER,), jnp.int32),
            pltpu.VMEM((IDX_PER_WORKER,), jnp.int32),
            pltpu.VMEM((LBUF,), jnp.float32),
            pltpu.VMEM((LBUF,), jnp.float32),
            pltpu.SemaphoreType.DMA,
            pltpu.SemaphoreType.DMA,
            pltpu.SemaphoreType.DMA,
        ],
    )(xf, tblf)
    return (out.reshape(SEQ, 8, NUM_WORKERS, 8, BBLK)
               .transpose(2, 4, 0, 1, 3)
               .reshape(BATCH, SEQ, N_PHONES))


# speedup vs baseline: 1.1286x; 1.1286x over previous
import jax
import jax.numpy as jnp
from jax import lax
from jax.experimental import pallas as pl
from jax.experimental.pallas import tpu as pltpu
from jax.experimental.pallas import tpu_sc as plsc

N_WORDS = 1000
N_PHONES = 64
IGNORE_IDX = -100
LANES = 16
NUM_CORES = 2
NUM_SUBCORES = 16
NUM_WORKERS = NUM_CORES * NUM_SUBCORES
BATCH = 4096
SEQ = 50
TOTAL_IDX = BATCH * SEQ
BBLK = BATCH // NUM_WORKERS
IDX_PER_WORKER = BBLK * SEQ
ROWS_PAD = 1008
LBUF = N_PHONES * BBLK
TILE = 8 * 128
P_TILES = N_PHONES // 8
L_STRIDE = 8 * 32 * 8 * 128
P_STRIDE = 32 * 8 * 128


def _tile_body(x_hbm, tbl_hbm, out_hbm,
               tbl_v, rnorm_v, idx_raw, idx_t, out_v0, out_v1,
               isem, osem0, osem1):
    cid = lax.axis_index("c")
    sid = lax.axis_index("s")
    wid = cid * NUM_SUBCORES + sid
    iota = lax.iota(jnp.int32, LANES)

    idx_cp = pltpu.async_copy(
        x_hbm.at[pl.ds(wid * IDX_PER_WORKER, IDX_PER_WORKER)], idx_raw, isem)
    pltpu.sync_copy(tbl_hbm, tbl_v.at[pl.ds(0, N_WORDS * N_PHONES)])

    zeros16 = jnp.zeros((LANES,), jnp.float32)
    for k in range((ROWS_PAD - N_WORDS) * N_PHONES // LANES):
        tbl_v[pl.ds(N_WORDS * N_PHONES + k * LANES, LANES)] = zeros16

    perm = [(iota + t) & (LANES - 1) for t in range(LANES)]

    def norm_group(k, carry):
        rows64 = (k * LANES + iota) * N_PHONES
        acc = [jnp.zeros((LANES,), jnp.float32) for _ in range(4)]
        for q in range(N_PHONES // LANES):
            rq = rows64 + q * LANES
            vs = [plsc.load_gather(tbl_v, [rq + perm[t]])
                  for t in range(LANES)]
            for t in range(LANES):
                acc[t % 4] = acc[t % 4] + vs[t]
        s = (acc[0] + acc[1]) + (acc[2] + acc[3])
        s = jnp.where(s > 0.0, s, 1.0)
        rnorm_v[pl.ds(k * LANES, LANES)] = 1.0 / s
        return carry

    lax.fori_loop(0, ROWS_PAD // LANES, norm_group, 0)
    idx_cp.wait()

    tr_base = [(g * LANES + iota) * SEQ for g in range(BBLK // LANES)]

    def tr_step(l, carry):
        for g in range(BBLK // LANES):
            raw = plsc.load_gather(idx_raw, [tr_base[g] + l])
            rows = jnp.where(raw == IGNORE_IDX, 0, raw)
            idx_t[pl.ds(l * BBLK + g * LANES, LANES)] = rows
        return carry

    lax.fori_loop(0, SEQ, tr_step, 0)

    def compute_l(l, out_buf):
        def group(g, carry):
            rows = idx_t[pl.ds(l * BBLK + g * LANES, LANES)]
            rn = plsc.load_gather(rnorm_v, [rows])
            rows64 = rows * N_PHONES
            g16 = g * LANES
            for q in range(N_PHONES // LANES):
                rq = rows64 + q * LANES
                sb = iota + (q * LANES * BBLK + g16)
                ps = [(iota + (t + g16)) & (LANES - 1)
                      for t in range(LANES)]
                vs = [plsc.load_gather(tbl_v, [rq + p]) for p in ps]
                for p, v in zip(ps, vs):
                    plsc.store_scatter(out_buf, [p * BBLK + sb], v * rn)
            return carry

        lax.fori_loop(0, BBLK // LANES, group, 0)

    def start_out(l, out_buf, sem):
        for P in range(P_TILES):
            pltpu.async_copy(
                out_buf.at[pl.ds(P * TILE, TILE)],
                out_hbm.at[pl.ds(l * L_STRIDE + P * P_STRIDE + wid * TILE, TILE)],
                sem)

    def drain(out_buf, sem):
        pltpu.make_async_copy(out_hbm.at[pl.ds(0, LBUF)], out_buf, sem).wait()

    compute_l(0, out_v0)
    start_out(0, out_v0, osem0)
    compute_l(1, out_v1)
    start_out(1, out_v1, osem1)

    def pair(p, carry):
        l0 = 2 * p
        drain(out_v0, osem0)
        compute_l(l0, out_v0)
        start_out(l0, out_v0, osem0)
        drain(out_v1, osem1)
        compute_l(l0 + 1, out_v1)
        start_out(l0 + 1, out_v1, osem1)
        return carry

    lax.fori_loop(1, SEQ // 2, pair, 0)

    drain(out_v0, osem0)
    drain(out_v1, osem1)


def kernel(x, pron_counts):
    xf = x.reshape(-1).astype(jnp.int32)
    tblf = pron_counts.reshape(-1)
    mesh = plsc.VectorSubcoreMesh(
        core_axis_name="c", subcore_axis_name="s",
        num_cores=NUM_CORES, num_subcores=NUM_SUBCORES)
    out = pl.kernel(
        _tile_body,
        out_type=jax.ShapeDtypeStruct((TOTAL_IDX * N_PHONES,), jnp.float32),
        mesh=mesh,
        compiler_params=pltpu.CompilerParams(needs_layout_passes=False),
        scratch_types=[
            pltpu.VMEM((ROWS_PAD * N_PHONES,), jnp.float32),
            pltpu.VMEM((ROWS_PAD,), jnp.float32),
            pltpu.VMEM((IDX_PER_WORKER,), jnp.int32),
            pltpu.VMEM((IDX_PER_WORKER,), jnp.int32),
            pltpu.VMEM((LBUF,), jnp.float32),
            pltpu.VMEM((LBUF,), jnp.float32),
            pltpu.SemaphoreType.DMA,
            pltpu.SemaphoreType.DMA,
            pltpu.SemaphoreType.DMA,
        ],
    )(xf, tblf)
    return (out.reshape(SEQ, 8, NUM_WORKERS, 8, BBLK)
               .transpose(2, 4, 0, 1, 3)
               .reshape(BATCH, SEQ, N_PHONES))
